# baseline (speedup 1.0000x reference)
import functools

import jax
import jax.numpy as jnp
from jax import lax
from jax.experimental import pallas as pl
from jax.experimental.pallas import tpu as pltpu
from jax.experimental.pallas import tpu_sc as plsc

_NC = 2
_NS = 16
_NW = _NC * _NS

_NBUF = 4


@functools.partial(jax.jit, static_argnames=("b", "s", "d"))
def _embed(x, table, b, s, d):
    rows_per_w = b // _NW
    nch = rows_per_w
    mesh = plsc.VectorSubcoreMesh(
        core_axis_name="c", subcore_axis_name="s",
        num_cores=_NC, num_subcores=_NS)

    @functools.partial(
        pl.kernel,
        out_type=jax.ShapeDtypeStruct((b * s, 2 * d), jnp.float32),
        mesh=mesh,
        scratch_types=[
            pltpu.VMEM((rows_per_w, s), jnp.int32),
            pltpu.VMEM((_NBUF, s, 2 * d), jnp.float32),
            pltpu.SemaphoreType.DMA((_NBUF,)),
            pltpu.SemaphoreType.DMA((_NBUF,)),
        ],
        compiler_params=pltpu.CompilerParams(use_tc_tiling_on_sc=False),
    )
    def k(x_hbm, table_hbm, out_hbm, idx_v, rows_v, gsem, wsem):
        wid = lax.axis_index("s") * _NC + lax.axis_index("c")
        row0 = wid * rows_per_w
        pltpu.sync_copy(x_hbm.at[pl.ds(row0, rows_per_w)], idx_v)

        def start_gather(j, bb):
            pltpu.async_copy(
                table_hbm.at[idx_v.at[j]], rows_v.at[bb], gsem.at[bb])

        def wait_gather(bb):
            pltpu.make_async_copy(
                table_hbm.at[pl.ds(0, s)], rows_v.at[bb], gsem.at[bb]
            ).wait()

        def start_write(j, bb):
            pltpu.async_copy(
                rows_v.at[bb],
                out_hbm.at[pl.ds((row0 + j) * s, s)],
                wsem.at[bb])

        def wait_write(bb):
            pltpu.make_async_copy(
                rows_v.at[bb], out_hbm.at[pl.ds(0, s)], wsem.at[bb]
            ).wait()

        for bb in range(_NBUF):
            start_gather(bb, bb)

        def body(jj, carry):
            j0 = jj * _NBUF
            for bb in range(_NBUF):
                wait_gather(bb)
                start_write(j0 + bb, bb)
            for bb in range(_NBUF):
                wait_write(bb)

                @pl.when(j0 + _NBUF + bb < nch)
                def _():
                    start_gather(j0 + _NBUF + bb, bb)

            return carry

        lax.fori_loop(0, nch // _NBUF, body, 0)

    return k(x, table)


def kernel(x, table):
    b, s = x.shape
    d = table.shape[1]
    tpad = jnp.pad(table, ((0, 0), (0, d)))
    out = _embed(x.astype(jnp.int32), tpad, b, s, d)
    return out.reshape(b, s, 2 * d)[:, :, :d]


# speedup vs baseline: 1.0925x; 1.0925x over previous
import functools

import jax
import jax.numpy as jnp
from jax import lax
from jax.experimental import pallas as pl
from jax.experimental.pallas import tpu as pltpu
from jax.experimental.pallas import tpu_sc as plsc

_NC = 2
_NS = 16
_NW = _NC * _NS

_NBUF = 8


@functools.partial(jax.jit, static_argnames=("b", "s", "d"))
def _embed(x, table, b, s, d):
    rows_per_w = b // _NW
    nch = rows_per_w
    mesh = plsc.VectorSubcoreMesh(
        core_axis_name="c", subcore_axis_name="s",
        num_cores=_NC, num_subcores=_NS)

    @functools.partial(
        pl.kernel,
        out_type=jax.ShapeDtypeStruct((b * s, 2 * d), jnp.float32),
        mesh=mesh,
        scratch_types=[
            pltpu.VMEM((rows_per_w, s), jnp.int32),
            pltpu.VMEM((_NBUF, s, d), jnp.float32),
            pltpu.SemaphoreType.DMA((_NBUF,)),
            pltpu.SemaphoreType.DMA((_NBUF,)),
        ],
        compiler_params=pltpu.CompilerParams(use_tc_tiling_on_sc=False),
    )
    def k(x_hbm, table_hbm, out_hbm, idx_v, rows_v, gsem, wsem):
        wid = lax.axis_index("s") * _NC + lax.axis_index("c")
        row0 = wid * rows_per_w
        pltpu.sync_copy(x_hbm.at[pl.ds(row0, rows_per_w)], idx_v)

        def start_gather(j, bb):
            pltpu.async_copy(
                table_hbm.at[idx_v.at[j]], rows_v.at[bb], gsem.at[bb])

        def wait_gather(bb):
            pltpu.make_async_copy(
                table_hbm.at[pl.ds(0, s)], rows_v.at[bb], gsem.at[bb]
            ).wait()

        def start_write(j, bb):
            pltpu.async_copy(
                rows_v.at[bb],
                out_hbm.at[pl.ds((row0 + j) * s, s), pl.ds(0, d)],
                wsem.at[bb])

        def wait_write(bb):
            pltpu.make_async_copy(
                rows_v.at[bb], out_hbm.at[pl.ds(0, s), pl.ds(0, d)],
                wsem.at[bb]
            ).wait()

        for bb in range(_NBUF):
            start_gather(bb, bb)

        def body(jj, carry):
            j0 = jj * _NBUF
            for bb in range(_NBUF):
                wait_gather(bb)
                start_write(j0 + bb, bb)
            for bb in range(_NBUF):
                wait_write(bb)

                @pl.when(j0 + _NBUF + bb < nch)
                def _():
                    start_gather(j0 + _NBUF + bb, bb)

            return carry

        lax.fori_loop(0, nch // _NBUF, body, 0)

    return k(x, table)


def kernel(x, table):
    b, s = x.shape
    d = table.shape[1]
    out = _embed(x.astype(jnp.int32), table, b, s, d)
    return out.reshape(b, s, 2 * d)[:, :, :d]
